# baseline (speedup 1.0000x reference)
import jax
import jax.numpy as jnp
from jax import lax
from jax.experimental import pallas as pl
from jax.experimental.pallas import tpu as pltpu
from jax.experimental.pallas import tpu_sc as plsc

BATCH = 16384
DIM = 16
_NC = 2
_NS = 16
_NW = _NC * _NS
_RPW = BATCH // _NW
_CHUNKS = _RPW // 16


def _mirt_body(stu_ref, exer_ref, thetas_ref, alphas_ref, betas_ref, out_ref,
               sidx, eidx, th, al, be, ov, sem):
    wid = lax.axis_index("s") * _NC + lax.axis_index("c")
    base = wid * _RPW
    pltpu.sync_copy(stu_ref.at[pl.ds(base, _RPW)], sidx)
    pltpu.sync_copy(exer_ref.at[pl.ds(base, _RPW)], eidx)
    c_th = pltpu.async_copy(thetas_ref.at[sidx], th, sem)
    c_al = pltpu.async_copy(alphas_ref.at[eidx], al, sem)
    c_be = pltpu.async_copy(betas_ref.at[eidx], be, sem)
    c_th.wait()
    c_al.wait()
    c_be.wait()

    lane = lax.iota(jnp.int32, 16)

    def chunk_body(k, carry):
        ridx = k * 16 + lane
        acc = jnp.zeros((16,), jnp.float32)
        for c in range(DIM):
            cv = jnp.full((16,), c, jnp.int32)
            acc = acc + plsc.load_gather(th, [ridx, cv]) * plsc.load_gather(
                al, [ridx, cv])
        off = pl.multiple_of(k * 16, 16)
        x = acc - be[pl.ds(off, 16)]
        ov[pl.ds(off, 16)] = 1.0 / (1.0 + jnp.exp(-x))
        return carry

    lax.fori_loop(0, _CHUNKS, chunk_body, 0)
    pltpu.sync_copy(ov, out_ref.at[pl.ds(base, _RPW)])


def kernel(stu_id, exer_id, kn_emb, thetas, alphas, betas):
    del kn_emb
    betas_flat = betas.reshape(-1)
    mesh = plsc.VectorSubcoreMesh(core_axis_name="c", subcore_axis_name="s",
                                  num_cores=_NC, num_subcores=_NS)
    return pl.kernel(
        _mirt_body,
        out_type=jax.ShapeDtypeStruct((BATCH,), jnp.float32),
        mesh=mesh,
        compiler_params=pltpu.CompilerParams(needs_layout_passes=False,
                                             use_tc_tiling_on_sc=False),
        scratch_types=[
            pltpu.VMEM((_RPW,), jnp.int32),
            pltpu.VMEM((_RPW,), jnp.int32),
            pltpu.VMEM((_RPW, DIM), jnp.float32),
            pltpu.VMEM((_RPW, DIM), jnp.float32),
            pltpu.VMEM((_RPW,), jnp.float32),
            pltpu.VMEM((_RPW,), jnp.float32),
            pltpu.SemaphoreType.DMA,
        ],
    )(stu_id, exer_id, thetas, alphas, betas_flat)
